# Initial kernel scaffold; baseline (speedup 1.0000x reference)
import jax
import jax.numpy as jnp
from jax.experimental import pallas as pl


def kernel(atomic_numbers, pos, edge_index, batch_indices, params):
    raise NotImplementedError("write your pallas kernel here")


# speedup vs baseline: 1.7222x; 1.7222x over previous
import functools

import jax
import jax.numpy as jnp
from jax import lax
from jax.experimental import pallas as pl
from jax.experimental.pallas import tpu as pltpu
from jax.experimental.pallas import tpu_sc as plsc

N = 50000
E = 800000
B = 64
H = 64
F = 64
G = 50
NI = 3
CUT = 10.0
ZMAX = 100

NC = 2
NS = 16
WIN = 1024
WPT = 50
WPT2 = 25
EP = NS * WIN * WPT
NX = 51200
NP = 50048
RPT = NP // NS
ZROWS = 184
SCP = pltpu.CompilerParams(use_tc_tiling_on_sc=False)


def _m8(v):
    return pl.multiple_of(v, 8)


def _ssp(x):
    return jax.nn.softplus(x) - jnp.log(2.0)


def _dist(ptab, row2, col2):
    mesh = plsc.VectorSubcoreMesh(core_axis_name="c", subcore_axis_name="s")

    @functools.partial(
        pl.kernel, mesh=mesh, compiler_params=SCP,
        out_type=jax.ShapeDtypeStruct((EP // 128, 128), jnp.float32),
        scratch_types=[
            pltpu.VMEM((WIN // 128, 128), jnp.int32),
            pltpu.VMEM((WIN // 128, 128), jnp.int32),
            pltpu.VMEM((1, 128), jnp.int32),
            pltpu.VMEM((6, 128), jnp.float32),
            pltpu.VMEM((WIN // 128, 128), jnp.float32),
            pltpu.VMEM_SHARED((3 * NX,), jnp.float32),
            pltpu.SemaphoreType.DMA,
        ],
    )
    def k(pt_h, row_h, col_h, out_h, idxr_v, idxc_v, idxs_v, g_v, d2_v,
          psp_s, sem):
        c = lax.axis_index("c")
        s = lax.axis_index("s")
        wid = s * NC + c

        @pl.when(s == 0)
        def _():
            pltpu.sync_copy(pt_h, psp_s)

        plsc.subcore_barrier()

        def body(w, _):
            rbase = _m8((wid * WPT2 + w) * (WIN // 128))
            pltpu.sync_copy(row_h.at[pl.ds(rbase, WIN // 128)], idxr_v)
            pltpu.sync_copy(col_h.at[pl.ds(rbase, WIN // 128)], idxc_v)

            def grp(j, _):
                for k6, (iv, off) in enumerate(
                    [(idxr_v, 0), (idxr_v, NX), (idxr_v, 2 * NX),
                     (idxc_v, 0), (idxc_v, NX), (idxc_v, 2 * NX)]
                ):
                    def sh(g2, _):
                        sl = pl.ds(g2 * 16, 16)
                        idxs_v[0, sl] = iv[j, sl] + off
                        return ()

                    lax.fori_loop(0, 8, sh, (), unroll=True)
                    pltpu.async_copy(
                        psp_s.at[idxs_v.at[0]], g_v.at[k6], sem
                    ).wait()

                def comp(g2, _):
                    sl = pl.ds(g2 * 16, 16)
                    dx = g_v[0, sl] - g_v[3, sl]
                    dy = g_v[1, sl] - g_v[4, sl]
                    dz = g_v[2, sl] - g_v[5, sl]
                    d2_v[j, sl] = dx * dx + dy * dy + dz * dz
                    return ()

                lax.fori_loop(0, 8, comp, (), unroll=True)
                return ()

            lax.fori_loop(0, WIN // 128, grp, (), unroll=False)
            pltpu.sync_copy(d2_v, out_h.at[pl.ds(rbase, WIN // 128)])
            return ()

        lax.fori_loop(0, WPT2, body, (), unroll=False)

    return k(ptab, row2, col2)


def _compact(xpad):
    mesh = plsc.VectorSubcoreMesh(core_axis_name="c", subcore_axis_name="s")
    CH = 200
    NCHUNK = 2 * N // CH

    @functools.partial(
        pl.kernel, mesh=mesh, compiler_params=SCP,
        out_type=jax.ShapeDtypeStruct((4 * N, 16), jnp.float32),
        scratch_types=[
            pltpu.VMEM((CH, 128), jnp.float32),
            pltpu.VMEM((CH, 16), jnp.float32),
            pltpu.VMEM((CH, 16), jnp.float32),
        ],
    )
    def k(xp_h, out_h, bp_v, ba_v, bb_v):
        c = lax.axis_index("c")
        s = lax.axis_index("s")
        wid = s * NC + c

        def body(kk, _):
            ch = wid + 32 * kk

            @pl.when(ch < NCHUNK)
            def _():
                st = _m8(ch * CH)
                cq = ch // (N // CH)
                pltpu.sync_copy(xp_h.at[pl.ds(st, CH)], bp_v)

                def cp(r, _):
                    ba_v[r, pl.ds(0, 16)] = bp_v[r, pl.ds(0, 16)]
                    bb_v[r, pl.ds(0, 16)] = bp_v[r, pl.ds(16, 16)]
                    return ()

                lax.fori_loop(0, CH, cp, (), unroll=8)
                pltpu.sync_copy(ba_v, out_h.at[pl.ds(_m8(st + cq * N), CH)])
                pltpu.sync_copy(bb_v, out_h.at[pl.ds(_m8(st + (cq + 1) * N), CH)])

            return ()

        lax.fori_loop(0, (NCHUNK + 31) // 32, body, (), unroll=False)

    return k(xpad)


def _conv_fn():
    mesh = plsc.VectorSubcoreMesh(core_axis_name="c", subcore_axis_name="s")

    @functools.partial(
        pl.kernel, mesh=mesh, compiler_params=SCP,
        out_type=[jax.ShapeDtypeStruct((2 * N, 128), jnp.float32),
                  jax.ShapeDtypeStruct((2 * N, 128), jnp.float32)],
        scratch_types=[
            pltpu.VMEM((WIN // 128, 128), jnp.int32),
            pltpu.VMEM((WIN // 128, 128), jnp.int32),
            pltpu.VMEM((WIN, 16), jnp.float32),
            pltpu.VMEM((WIN // 8, 128), jnp.float32),
            pltpu.VMEM((ZROWS, 16), jnp.float32),
            pltpu.VMEM((ZROWS, 128), jnp.float32),
            pltpu.VMEM_SHARED((NP, 16), jnp.float32),
            pltpu.SemaphoreType.DMA,
        ],
    )
    def k(x_h, wf_h, row_h, col_h, outa_h, outb_h, idxr_v, idxc_v, xr_v,
          wfb_v, z_v, b128_v, agg_s, sem):
        c = lax.axis_index("c")
        s = lax.axis_index("s")

        for half, out_h in ((0, outa_h), (1, outb_h)):
            def zb(r, _):
                z_v[r, pl.ds(0, 16)] = jnp.zeros((16,), jnp.float32)
                return ()

            lax.fori_loop(0, ZROWS, zb, (), unroll=8)

            def zc(j, _):
                pltpu.sync_copy(
                    z_v, agg_s.at[pl.ds(_m8(s * RPT + j * ZROWS), ZROWS)])
                return ()

            lax.fori_loop(0, RPT // ZROWS, zc, (), unroll=False)
            plsc.subcore_barrier()

            qoff = (2 * c + half) * N
            wfoff = (2 * c + half) * (EP // 8)

            def body(w, _):
                base = pl.multiple_of((s * WPT + w) * WIN, WIN)
                rbase = _m8((s * WPT + w) * (WIN // 128))
                pltpu.sync_copy(row_h.at[pl.ds(rbase, WIN // 128)], idxr_v)
                pltpu.sync_copy(col_h.at[pl.ds(rbase, WIN // 128)], idxc_v)

                def addoff(r, _):
                    for g in range(8):
                        sl = pl.ds(g * 16, 16)
                        idxr_v[r, sl] = idxr_v[r, sl] + qoff
                    return ()

                lax.fori_loop(0, WIN // 128, addoff, (), unroll=True)

                cps = [
                    pltpu.async_copy(
                        x_h.at[idxr_v.at[j]],
                        xr_v.at[pl.ds(j * 128, 128)],
                        sem,
                    )
                    for j in range(WIN // 128)
                ]
                for cp in cps:
                    cp.wait()
                pltpu.sync_copy(
                    wf_h.at[pl.ds(
                        pl.multiple_of(wfoff + base // 8, WIN // 8),
                        WIN // 8)],
                    wfb_v,
                )

                def mul(q2, _):
                    for j in range(8):
                        sa = pl.ds(0, 16)
                        sb = pl.ds(j * 16, 16)
                        r = 8 * q2 + j
                        xr_v[r, sa] = xr_v[r, sa] * wfb_v[q2, sb]
                    return ()

                lax.fori_loop(0, WIN // 8, mul, (), unroll=4)

                for j in range(WIN // 128):
                    pltpu.sync_copy(
                        xr_v.at[pl.ds(j * 128, 128)],
                        agg_s.at[idxc_v.at[j]],
                        add=True,
                    )
                return ()

            lax.fori_loop(0, WPT, body, (), unroll=False)

            plsc.subcore_barrier()

            def flch(rows, st):
                pltpu.sync_copy(agg_s.at[pl.ds(_m8(st), rows)],
                                z_v.at[pl.ds(0, rows)])

                def fp(r, _):
                    b128_v[r, pl.ds(0, 16)] = z_v[r, pl.ds(0, 16)]
                    return ()

                lax.fori_loop(0, rows, fp, (), unroll=8)
                pltpu.sync_copy(
                    b128_v.at[pl.ds(0, rows)],
                    out_h.at[pl.ds(_m8(c * N + st), rows)],
                )

            nfl = jnp.where(s == 15, 16, 17)

            def fl(kk, _):
                flch(ZROWS, s * RPT + kk * ZROWS)
                return ()

            lax.fori_loop(0, nfl, fl, (), unroll=False)

            @pl.when(s == 15)
            def _():
                flch(136, 15 * RPT + 16 * ZROWS)

            plsc.subcore_barrier()

    return k


def _embed_kernel(z_ref, emb_ref, lin1_ref, h_ref, x_ref):
    Bn = z_ref.shape[0]
    z = z_ref[...]
    zo = (z == lax.broadcasted_iota(jnp.int32, (1, ZMAX), 1)).astype(jnp.float32)
    h = jnp.dot(zo, emb_ref[...], preferred_element_type=jnp.float32)
    h_ref[...] = h
    x = jnp.dot(h, lin1_ref[...], preferred_element_type=jnp.float32)
    zpad = jnp.zeros((Bn, 96), jnp.float32)
    x_ref[0] = jnp.concatenate([x[:, :32], zpad], axis=1)
    x_ref[1] = jnp.concatenate([x[:, 32:], zpad], axis=1)


def _embed(z, emb, lin1):
    Bn = 2000
    grid = N // Bn
    return pl.pallas_call(
        _embed_kernel,
        grid=(grid,),
        in_specs=[
            pl.BlockSpec((Bn, 1), lambda i: (i, 0)),
            pl.BlockSpec((ZMAX, H), lambda i: (0, 0)),
            pl.BlockSpec((H, H), lambda i: (0, 0)),
        ],
        out_specs=[
            pl.BlockSpec((Bn, H), lambda i: (i, 0)),
            pl.BlockSpec((2, Bn, 128), lambda i: (0, i, 0)),
        ],
        out_shape=[
            jax.ShapeDtypeStruct((N, H), jnp.float32),
            jax.ShapeDtypeStruct((2, N, 128), jnp.float32),
        ],
    )(z.reshape(N, 1), emb, lin1)


def _filter_kernel(d2_ref, w1_ref, b1_ref, w2_ref, b2_ref, *out_refs):
    i = pl.program_id(0)
    Be = 2048
    a = d2_ref[...]
    sidx = lax.broadcasted_iota(jnp.int32, (Be, 1), 0)
    lsel = (sidx // 128 == lax.broadcasted_iota(jnp.int32, (1, 16), 1)
            ).astype(jnp.float32)
    m = jnp.dot(lsel, a, preferred_element_type=jnp.float32)
    mask = (sidx % 128 == lax.broadcasted_iota(jnp.int32, (1, 128), 1)
            ).astype(jnp.float32)
    d2 = jnp.sum(m * mask, axis=1, keepdims=True)
    d = jnp.sqrt(d2 + 1e-12)
    step = CUT / (G - 1)
    offs = lax.broadcasted_iota(jnp.int32, (1, G), 1).astype(jnp.float32) * step
    coeff = -0.5 / (step * step)
    rbf = jnp.exp(coeff * (d - offs) ** 2)
    ccut = 0.5 * (jnp.cos(d * (jnp.pi / CUT)) + 1.0) * (d < CUT).astype(jnp.float32)
    gid = i * Be + sidx
    valid = (gid < E).astype(jnp.float32)
    scale = ccut * valid
    for b in range(NI):
        t = jnp.dot(rbf, w1_ref[b], preferred_element_type=jnp.float32) + b1_ref[b]
        w = jnp.dot(_ssp(t), w2_ref[b], preferred_element_type=jnp.float32) + b2_ref[b]
        wf = w * scale
        w3 = wf.reshape(Be // 8, 8, 64)
        for q in range(4):
            parts = [w3[:, j, 16 * q:16 * q + 16] for j in range(8)]
            out_refs[0][b, q] = jnp.concatenate(parts, axis=1)


def _filters(d2v, w1s, b1s, w2s, b2s):
    Be = 2048
    grid = EP // Be
    return pl.pallas_call(
        _filter_kernel,
        grid=(grid,),
        in_specs=[
            pl.BlockSpec((16, 128), lambda i: (i, 0)),
            pl.BlockSpec((NI, G, F), lambda i: (0, 0, 0)),
            pl.BlockSpec((NI, 1, F), lambda i: (0, 0, 0)),
            pl.BlockSpec((NI, F, F), lambda i: (0, 0, 0)),
            pl.BlockSpec((NI, 1, F), lambda i: (0, 0, 0)),
        ],
        out_specs=pl.BlockSpec((NI, 4, Be // 8, 128), lambda i: (0, 0, i, 0)),
        out_shape=jax.ShapeDtypeStruct((NI, 4, EP // 8, 128), jnp.float32),
    )(d2v, w1s, b1s, w2s, b2s)


def _update_kernel(a0_ref, b0_ref, a1_ref, b1_ref, h_ref, lin2w_ref,
                   lin2b_ref, linw_ref, linb_ref, nxt_ref, hout_ref, x_ref):
    Bn = h_ref.shape[0]
    agg = jnp.concatenate(
        [a0_ref[...][:, :16], b0_ref[...][:, :16],
         a1_ref[...][:, :16], b1_ref[...][:, :16]], axis=1)
    t = jnp.dot(agg, lin2w_ref[...], preferred_element_type=jnp.float32) + lin2b_ref[...]
    out = jnp.dot(_ssp(t), linw_ref[...], preferred_element_type=jnp.float32) + linb_ref[...]
    h = h_ref[...] + out
    hout_ref[...] = h
    x = jnp.dot(h, nxt_ref[...], preferred_element_type=jnp.float32)
    zpad = jnp.zeros((Bn, 96), jnp.float32)
    x_ref[0] = jnp.concatenate([x[:, :32], zpad], axis=1)
    x_ref[1] = jnp.concatenate([x[:, 32:], zpad], axis=1)


def _update(aggA, aggB, h, lin2w, lin2b, linw, linb, lin1_next):
    Bn = 2000
    grid = N // Bn
    return pl.pallas_call(
        _update_kernel,
        grid=(grid,),
        in_specs=[
            pl.BlockSpec((Bn, 128), lambda i: (i, 0)),
            pl.BlockSpec((Bn, 128), lambda i: (i, 0)),
            pl.BlockSpec((Bn, 128), lambda i: (i + N // Bn, 0)),
            pl.BlockSpec((Bn, 128), lambda i: (i + N // Bn, 0)),
            pl.BlockSpec((Bn, H), lambda i: (i, 0)),
            pl.BlockSpec((H, H), lambda i: (0, 0)),
            pl.BlockSpec((1, H), lambda i: (0, 0)),
            pl.BlockSpec((H, H), lambda i: (0, 0)),
            pl.BlockSpec((1, H), lambda i: (0, 0)),
            pl.BlockSpec((H, H), lambda i: (0, 0)),
        ],
        out_specs=[
            pl.BlockSpec((Bn, H), lambda i: (i, 0)),
            pl.BlockSpec((2, Bn, 128), lambda i: (0, i, 0)),
        ],
        out_shape=[
            jax.ShapeDtypeStruct((N, H), jnp.float32),
            jax.ShapeDtypeStruct((2, N, 128), jnp.float32),
        ],
    )(aggA, aggB, aggA, aggB, h, lin2w, lin2b, linw, linb, lin1_next)


def _final_kernel(h_ref, o1w_ref, o1b_ref, o2w_ref, o2b_ref,
                  bat_ref, en_ref):
    i = pl.program_id(0)
    h = h_ref[...]
    hh = _ssp(jnp.dot(h, o1w_ref[...], preferred_element_type=jnp.float32) + o1b_ref[...])
    e = jnp.dot(hh, o2w_ref[...], preferred_element_type=jnp.float32) + o2b_ref[...]
    onehot = (bat_ref[...] == lax.broadcasted_iota(jnp.int32, (1, B), 1)).astype(jnp.float32)
    contrib = jnp.sum(e * onehot, axis=0, keepdims=True)

    @pl.when(i == 0)
    def _():
        en_ref[...] = jnp.zeros_like(en_ref)

    en_ref[...] += contrib


def _final(h, o1w, o1b, o2w, o2b, batch):
    Bn = 2000
    grid = N // Bn
    return pl.pallas_call(
        _final_kernel,
        grid=(grid,),
        in_specs=[
            pl.BlockSpec((Bn, H), lambda i: (i, 0)),
            pl.BlockSpec((H, H // 2), lambda i: (0, 0)),
            pl.BlockSpec((1, H // 2), lambda i: (0, 0)),
            pl.BlockSpec((H // 2, 1), lambda i: (0, 0)),
            pl.BlockSpec((1, 1), lambda i: (0, 0)),
            pl.BlockSpec((Bn, 1), lambda i: (i, 0)),
        ],
        out_specs=pl.BlockSpec((1, B), lambda i: (0, 0)),
        out_shape=jax.ShapeDtypeStruct((1, B), jnp.float32),
    )(h, o1w, o1b, o2w, o2b, batch.reshape(N, 1))


def kernel(atomic_numbers, pos, edge_index, batch_indices, params):
    f32 = jnp.float32
    zpad = jnp.zeros((NX - N,), f32)
    ptab = jnp.concatenate([pos[:, 0], zpad, pos[:, 1], zpad, pos[:, 2], zpad])
    row = edge_index[0].astype(jnp.int32)
    col = edge_index[1].astype(jnp.int32)
    npad = EP - E
    padi = (jnp.arange(npad, dtype=jnp.int32) * 37) % N
    row2 = jnp.concatenate([row, padi]).reshape(EP // 128, 128)
    col2 = jnp.concatenate([col, padi]).reshape(EP // 128, 128)

    inter = params["interactions"]
    w1s = jnp.stack([blk["mlp_w1"] for blk in inter])
    b1s = jnp.stack([blk["mlp_b1"] for blk in inter]).reshape(NI, 1, F)
    w2s = jnp.stack([blk["mlp_w2"] for blk in inter])
    b2s = jnp.stack([blk["mlp_b2"] for blk in inter]).reshape(NI, 1, F)

    d2v = _dist(ptab, row2, col2)
    wf_all = _filters(d2v, w1s, b1s, w2s, b2s).reshape(NI, 4 * (EP // 8), 128)

    conv = _conv_fn()
    h, xpad = _embed(atomic_numbers.astype(jnp.int32), params["embedding"],
                     inter[0]["lin1_w"])
    lin2w_s = jnp.stack([blk["lin2_w"] for blk in inter])
    lin2b_s = jnp.stack([blk["lin2_b"] for blk in inter]).reshape(NI, 1, H)
    linw_s = jnp.stack([blk["lin_w"] for blk in inter])
    linb_s = jnp.stack([blk["lin_b"] for blk in inter]).reshape(NI, 1, H)
    lin1n_s = jnp.stack([inter[(b2 + 1) % NI]["lin1_w"] for b2 in range(NI)])

    def body(carry, xs):
        h_c, xpad_c = carry
        wf_b, l2w, l2b, lw, lb, l1n = xs
        xc = _compact(xpad_c.reshape(2 * N, 128))
        aggA, aggB = conv(xc, wf_b, row2, col2)
        h_n, xpad_n = _update(aggA, aggB, h_c, l2w, l2b, lw, lb, l1n)
        return (h_n, xpad_n), ()

    (h, xpad), _ = lax.scan(
        body, (h, xpad),
        (wf_all, lin2w_s, lin2b_s, linw_s, linb_s, lin1n_s))

    en = _final(h, params["out1_w"], params["out1_b"].reshape(1, H // 2),
                params["out2_w"], params["out2_b"].reshape(1, 1),
                batch_indices.astype(jnp.int32))
    energies = en.reshape(B)
    forces = jnp.zeros((N, 3), f32)
    stress = jnp.zeros((B, 3, 3), f32)
    features = jnp.zeros((N, H), f32)
    return (energies, forces, stress, features)


# speedup vs baseline: 1.8082x; 1.0499x over previous
import functools

import jax
import jax.numpy as jnp
from jax import lax
from jax.experimental import pallas as pl
from jax.experimental.pallas import tpu as pltpu
from jax.experimental.pallas import tpu_sc as plsc

N = 50000
E = 800000
B = 64
H = 64
F = 64
G = 50
NI = 3
CUT = 10.0
ZMAX = 100

NC = 2
NS = 16
WIN = 1024
WPT = 50
WPT2 = 25
EP = NS * WIN * WPT
NX = 51200
NP = 50048
RPT = NP // NS
ZROWS = 184
SCP = pltpu.CompilerParams(use_tc_tiling_on_sc=False)


def _m8(v):
    return pl.multiple_of(v, 8)


def _ssp(x):
    return jax.nn.softplus(x) - jnp.log(2.0)


def _dist(ptab, row2, col2):
    mesh = plsc.VectorSubcoreMesh(core_axis_name="c", subcore_axis_name="s")

    @functools.partial(
        pl.kernel, mesh=mesh, compiler_params=SCP,
        out_type=jax.ShapeDtypeStruct((EP // 128, 128), jnp.float32),
        scratch_types=[
            pltpu.VMEM((WIN // 128, 128), jnp.int32),
            pltpu.VMEM((WIN // 128, 128), jnp.int32),
            pltpu.VMEM((1, 128), jnp.int32),
            pltpu.VMEM((6, 128), jnp.float32),
            pltpu.VMEM((WIN // 128, 128), jnp.float32),
            pltpu.VMEM_SHARED((3 * NX,), jnp.float32),
            pltpu.SemaphoreType.DMA,
        ],
    )
    def k(pt_h, row_h, col_h, out_h, idxr_v, idxc_v, idxs_v, g_v, d2_v,
          psp_s, sem):
        c = lax.axis_index("c")
        s = lax.axis_index("s")
        wid = s * NC + c

        @pl.when(s == 0)
        def _():
            pltpu.sync_copy(pt_h, psp_s)

        plsc.subcore_barrier()

        def body(w, _):
            rbase = _m8((wid * WPT2 + w) * (WIN // 128))
            pltpu.sync_copy(row_h.at[pl.ds(rbase, WIN // 128)], idxr_v)
            pltpu.sync_copy(col_h.at[pl.ds(rbase, WIN // 128)], idxc_v)

            def grp(j, _):
                for k6, (iv, off) in enumerate(
                    [(idxr_v, 0), (idxr_v, NX), (idxr_v, 2 * NX),
                     (idxc_v, 0), (idxc_v, NX), (idxc_v, 2 * NX)]
                ):
                    def sh(g2, _):
                        sl = pl.ds(g2 * 16, 16)
                        idxs_v[0, sl] = iv[j, sl] + off
                        return ()

                    lax.fori_loop(0, 8, sh, (), unroll=True)
                    pltpu.async_copy(
                        psp_s.at[idxs_v.at[0]], g_v.at[k6], sem
                    ).wait()

                def comp(g2, _):
                    sl = pl.ds(g2 * 16, 16)
                    dx = g_v[0, sl] - g_v[3, sl]
                    dy = g_v[1, sl] - g_v[4, sl]
                    dz = g_v[2, sl] - g_v[5, sl]
                    d2_v[j, sl] = dx * dx + dy * dy + dz * dz
                    return ()

                lax.fori_loop(0, 8, comp, (), unroll=True)
                return ()

            lax.fori_loop(0, WIN // 128, grp, (), unroll=False)
            pltpu.sync_copy(d2_v, out_h.at[pl.ds(rbase, WIN // 128)])
            return ()

        lax.fori_loop(0, WPT2, body, (), unroll=False)

    return k(ptab, row2, col2)


def _compact(xpad):
    mesh = plsc.VectorSubcoreMesh(core_axis_name="c", subcore_axis_name="s")
    CH = 200
    NCHUNK = 2 * N // CH

    @functools.partial(
        pl.kernel, mesh=mesh, compiler_params=SCP,
        out_type=jax.ShapeDtypeStruct((4 * N, 16), jnp.float32),
        scratch_types=[
            pltpu.VMEM((CH, 128), jnp.float32),
            pltpu.VMEM((CH, 16), jnp.float32),
            pltpu.VMEM((CH, 16), jnp.float32),
        ],
    )
    def k(xp_h, out_h, bp_v, ba_v, bb_v):
        c = lax.axis_index("c")
        s = lax.axis_index("s")
        wid = s * NC + c

        def body(kk, _):
            ch = wid + 32 * kk

            @pl.when(ch < NCHUNK)
            def _():
                st = _m8(ch * CH)
                cq = ch // (N // CH)
                pltpu.sync_copy(xp_h.at[pl.ds(st, CH)], bp_v)

                def cp(r, _):
                    ba_v[r, pl.ds(0, 16)] = bp_v[r, pl.ds(0, 16)]
                    bb_v[r, pl.ds(0, 16)] = bp_v[r, pl.ds(16, 16)]
                    return ()

                lax.fori_loop(0, CH, cp, (), unroll=8)
                pltpu.sync_copy(ba_v, out_h.at[pl.ds(_m8(st + cq * N), CH)])
                pltpu.sync_copy(bb_v, out_h.at[pl.ds(_m8(st + (cq + 1) * N), CH)])

            return ()

        lax.fori_loop(0, (NCHUNK + 31) // 32, body, (), unroll=False)

    return k(xpad)


def _conv_fn():
    mesh = plsc.VectorSubcoreMesh(core_axis_name="c", subcore_axis_name="s")

    @functools.partial(
        pl.kernel, mesh=mesh, compiler_params=SCP,
        out_type=[jax.ShapeDtypeStruct((2 * N, 128), jnp.float32),
                  jax.ShapeDtypeStruct((2 * N, 128), jnp.float32)],
        scratch_types=[
            pltpu.VMEM((WIN // 128, 128), jnp.int32),
            pltpu.VMEM((WIN // 128, 128), jnp.int32),
            pltpu.VMEM((WIN, 16), jnp.float32),
            pltpu.VMEM((WIN // 8, 128), jnp.float32),
            pltpu.VMEM((ZROWS, 16), jnp.float32),
            pltpu.VMEM((ZROWS, 128), jnp.float32),
            pltpu.VMEM_SHARED((NP, 16), jnp.float32),
            pltpu.SemaphoreType.DMA,
        ],
    )
    def k(x_h, wf_h, row_h, col_h, outa_h, outb_h, idxr_v, idxc_v, xr_v,
          wfb_v, z_v, b128_v, agg_s, sem):
        c = lax.axis_index("c")
        s = lax.axis_index("s")

        for half, out_h in ((0, outa_h), (1, outb_h)):
            def zb(r, _):
                z_v[r, pl.ds(0, 16)] = jnp.zeros((16,), jnp.float32)
                return ()

            lax.fori_loop(0, ZROWS, zb, (), unroll=8)

            def zc(j, _):
                pltpu.sync_copy(
                    z_v, agg_s.at[pl.ds(_m8(s * RPT + j * ZROWS), ZROWS)])
                return ()

            lax.fori_loop(0, RPT // ZROWS, zc, (), unroll=False)
            plsc.subcore_barrier()

            qoff = (2 * c + half) * N
            wfoff = (2 * c + half) * (EP // 8)

            def body(w, _):
                base = pl.multiple_of((s * WPT + w) * WIN, WIN)
                rbase = _m8((s * WPT + w) * (WIN // 128))
                pltpu.sync_copy(row_h.at[pl.ds(rbase, WIN // 128)], idxr_v)
                pltpu.sync_copy(col_h.at[pl.ds(rbase, WIN // 128)], idxc_v)

                def addoff(r, _):
                    for g in range(8):
                        sl = pl.ds(g * 16, 16)
                        idxr_v[r, sl] = idxr_v[r, sl] + qoff
                    return ()

                lax.fori_loop(0, WIN // 128, addoff, (), unroll=True)

                cps = [
                    pltpu.async_copy(
                        x_h.at[idxr_v.at[j]],
                        xr_v.at[pl.ds(j * 128, 128)],
                        sem,
                    )
                    for j in range(WIN // 128)
                ]
                for cp in cps:
                    cp.wait()
                pltpu.sync_copy(
                    wf_h.at[pl.ds(
                        pl.multiple_of(wfoff + base // 8, WIN // 8),
                        WIN // 8)],
                    wfb_v,
                )

                def mul(q2, _):
                    for j in range(8):
                        sa = pl.ds(0, 16)
                        sb = pl.ds(j * 16, 16)
                        r = 8 * q2 + j
                        xr_v[r, sa] = xr_v[r, sa] * wfb_v[q2, sb]
                    return ()

                lax.fori_loop(0, WIN // 8, mul, (), unroll=4)

                for j in range(WIN // 128):
                    pltpu.sync_copy(
                        xr_v.at[pl.ds(j * 128, 128)],
                        agg_s.at[idxc_v.at[j]],
                        add=True,
                    )
                return ()

            lax.fori_loop(0, WPT, body, (), unroll=False)

            plsc.subcore_barrier()

            def flch(rows, st):
                pltpu.sync_copy(agg_s.at[pl.ds(_m8(st), rows)],
                                z_v.at[pl.ds(0, rows)])

                def fp(r, _):
                    b128_v[r, pl.ds(0, 16)] = z_v[r, pl.ds(0, 16)]
                    return ()

                lax.fori_loop(0, rows, fp, (), unroll=8)
                pltpu.sync_copy(
                    b128_v.at[pl.ds(0, rows)],
                    out_h.at[pl.ds(_m8(c * N + st), rows)],
                )

            nfl = jnp.where(s == 15, 16, 17)

            def fl(kk, _):
                flch(ZROWS, s * RPT + kk * ZROWS)
                return ()

            lax.fori_loop(0, nfl, fl, (), unroll=False)

            @pl.when(s == 15)
            def _():
                flch(136, 15 * RPT + 16 * ZROWS)

            plsc.subcore_barrier()

    return k


def _embed_kernel(z_ref, emb_ref, lin1_ref, h_ref, x_ref):
    Bn = z_ref.shape[0]
    z = z_ref[...]
    zo = (z == lax.broadcasted_iota(jnp.int32, (1, ZMAX), 1)).astype(jnp.float32)
    h = jnp.dot(zo, emb_ref[...], preferred_element_type=jnp.float32)
    h_ref[...] = h
    x = jnp.dot(h, lin1_ref[...], preferred_element_type=jnp.float32)
    zpad = jnp.zeros((Bn, 96), jnp.float32)
    x_ref[0] = jnp.concatenate([x[:, :32], zpad], axis=1)
    x_ref[1] = jnp.concatenate([x[:, 32:], zpad], axis=1)


def _embed(z, emb, lin1):
    Bn = 2000
    grid = N // Bn
    return pl.pallas_call(
        _embed_kernel,
        grid=(grid,),
        in_specs=[
            pl.BlockSpec((Bn, 1), lambda i: (i, 0)),
            pl.BlockSpec((ZMAX, H), lambda i: (0, 0)),
            pl.BlockSpec((H, H), lambda i: (0, 0)),
        ],
        out_specs=[
            pl.BlockSpec((Bn, H), lambda i: (i, 0)),
            pl.BlockSpec((2, Bn, 128), lambda i: (0, i, 0)),
        ],
        out_shape=[
            jax.ShapeDtypeStruct((N, H), jnp.float32),
            jax.ShapeDtypeStruct((2, N, 128), jnp.float32),
        ],
    )(z.reshape(N, 1), emb, lin1)


def _filter_kernel(d2_ref, w1_ref, b1_ref, w2_ref, b2_ref, out_ref):
    i = pl.program_id(0)
    Be = 2048
    a = d2_ref[...]
    dc = jnp.sqrt(a + 1e-12)
    ccutc = 0.5 * (jnp.cos(dc * (jnp.pi / CUT)) + 1.0) * (dc < CUT).astype(jnp.float32)
    gidc = (i * Be
            + lax.broadcasted_iota(jnp.int32, (16, 128), 0) * 128
            + lax.broadcasted_iota(jnp.int32, (16, 128), 1))
    scalec = ccutc * (gidc < E).astype(jnp.float32)
    sidx = lax.broadcasted_iota(jnp.int32, (Be, 1), 0)
    lsel = (sidx // 128 == lax.broadcasted_iota(jnp.int32, (1, 16), 1)
            ).astype(jnp.float32)
    mask = (sidx % 128 == lax.broadcasted_iota(jnp.int32, (1, 128), 1)
            ).astype(jnp.float32)
    md = jnp.dot(lsel, dc, preferred_element_type=jnp.float32)
    ms = jnp.dot(lsel, scalec, preferred_element_type=jnp.float32)
    d = jnp.sum(md * mask, axis=1, keepdims=True)
    scale = jnp.sum(ms * mask, axis=1, keepdims=True)
    step = CUT / (G - 1)
    offs = lax.broadcasted_iota(jnp.int32, (1, G), 1).astype(jnp.float32) * step
    coeff = -0.5 / (step * step)
    rbf = jnp.exp(coeff * (d - offs) ** 2)
    t = jnp.dot(rbf, w1_ref[...], preferred_element_type=jnp.float32) + b1_ref[...]
    w = jnp.dot(_ssp(t), w2_ref[...], preferred_element_type=jnp.float32) + b2_ref[...]
    wf = w * scale
    w3 = wf.reshape(Be // 8, 8, 64)
    for q in range(4):
        parts = [w3[:, j, 16 * q:16 * q + 16] for j in range(8)]
        out_ref[q] = jnp.concatenate(parts, axis=1)


def _filter1(d2v, w1, b1, w2, b2):
    Be = 2048
    grid = EP // Be
    return pl.pallas_call(
        _filter_kernel,
        grid=(grid,),
        in_specs=[
            pl.BlockSpec((16, 128), lambda i: (i, 0)),
            pl.BlockSpec((G, F), lambda i: (0, 0)),
            pl.BlockSpec((1, F), lambda i: (0, 0)),
            pl.BlockSpec((F, F), lambda i: (0, 0)),
            pl.BlockSpec((1, F), lambda i: (0, 0)),
        ],
        out_specs=pl.BlockSpec((4, Be // 8, 128), lambda i: (0, i, 0)),
        out_shape=jax.ShapeDtypeStruct((4, EP // 8, 128), jnp.float32),
    )(d2v, w1, b1, w2, b2).reshape(4 * (EP // 8), 128)


def _update_kernel(a0_ref, b0_ref, a1_ref, b1_ref, h_ref, lin2w_ref,
                   lin2b_ref, linw_ref, linb_ref, nxt_ref, hout_ref, x_ref):
    Bn = h_ref.shape[0]
    agg = jnp.concatenate(
        [a0_ref[...][:, :16], b0_ref[...][:, :16],
         a1_ref[...][:, :16], b1_ref[...][:, :16]], axis=1)
    t = jnp.dot(agg, lin2w_ref[...], preferred_element_type=jnp.float32) + lin2b_ref[...]
    out = jnp.dot(_ssp(t), linw_ref[...], preferred_element_type=jnp.float32) + linb_ref[...]
    h = h_ref[...] + out
    hout_ref[...] = h
    x = jnp.dot(h, nxt_ref[...], preferred_element_type=jnp.float32)
    zpad = jnp.zeros((Bn, 96), jnp.float32)
    x_ref[0] = jnp.concatenate([x[:, :32], zpad], axis=1)
    x_ref[1] = jnp.concatenate([x[:, 32:], zpad], axis=1)


def _update(aggA, aggB, h, lin2w, lin2b, linw, linb, lin1_next):
    Bn = 2000
    grid = N // Bn
    return pl.pallas_call(
        _update_kernel,
        grid=(grid,),
        in_specs=[
            pl.BlockSpec((Bn, 128), lambda i: (i, 0)),
            pl.BlockSpec((Bn, 128), lambda i: (i, 0)),
            pl.BlockSpec((Bn, 128), lambda i: (i + N // Bn, 0)),
            pl.BlockSpec((Bn, 128), lambda i: (i + N // Bn, 0)),
            pl.BlockSpec((Bn, H), lambda i: (i, 0)),
            pl.BlockSpec((H, H), lambda i: (0, 0)),
            pl.BlockSpec((1, H), lambda i: (0, 0)),
            pl.BlockSpec((H, H), lambda i: (0, 0)),
            pl.BlockSpec((1, H), lambda i: (0, 0)),
            pl.BlockSpec((H, H), lambda i: (0, 0)),
        ],
        out_specs=[
            pl.BlockSpec((Bn, H), lambda i: (i, 0)),
            pl.BlockSpec((2, Bn, 128), lambda i: (0, i, 0)),
        ],
        out_shape=[
            jax.ShapeDtypeStruct((N, H), jnp.float32),
            jax.ShapeDtypeStruct((2, N, 128), jnp.float32),
        ],
    )(aggA, aggB, aggA, aggB, h, lin2w, lin2b, linw, linb, lin1_next)


def _final_kernel(h_ref, o1w_ref, o1b_ref, o2w_ref, o2b_ref,
                  bat_ref, en_ref):
    i = pl.program_id(0)
    h = h_ref[...]
    hh = _ssp(jnp.dot(h, o1w_ref[...], preferred_element_type=jnp.float32) + o1b_ref[...])
    e = jnp.dot(hh, o2w_ref[...], preferred_element_type=jnp.float32) + o2b_ref[...]
    onehot = (bat_ref[...] == lax.broadcasted_iota(jnp.int32, (1, B), 1)).astype(jnp.float32)
    contrib = jnp.sum(e * onehot, axis=0, keepdims=True)

    @pl.when(i == 0)
    def _():
        en_ref[...] = jnp.zeros_like(en_ref)

    en_ref[...] += contrib


def _final(h, o1w, o1b, o2w, o2b, batch):
    Bn = 2000
    grid = N // Bn
    return pl.pallas_call(
        _final_kernel,
        grid=(grid,),
        in_specs=[
            pl.BlockSpec((Bn, H), lambda i: (i, 0)),
            pl.BlockSpec((H, H // 2), lambda i: (0, 0)),
            pl.BlockSpec((1, H // 2), lambda i: (0, 0)),
            pl.BlockSpec((H // 2, 1), lambda i: (0, 0)),
            pl.BlockSpec((1, 1), lambda i: (0, 0)),
            pl.BlockSpec((Bn, 1), lambda i: (i, 0)),
        ],
        out_specs=pl.BlockSpec((1, B), lambda i: (0, 0)),
        out_shape=jax.ShapeDtypeStruct((1, B), jnp.float32),
    )(h, o1w, o1b, o2w, o2b, batch.reshape(N, 1))


def kernel(atomic_numbers, pos, edge_index, batch_indices, params):
    f32 = jnp.float32
    zpad = jnp.zeros((NX - N,), f32)
    ptab = jnp.concatenate([pos[:, 0], zpad, pos[:, 1], zpad, pos[:, 2], zpad])
    row = edge_index[0].astype(jnp.int32)
    col = edge_index[1].astype(jnp.int32)
    npad = EP - E
    padi = (jnp.arange(npad, dtype=jnp.int32) * 37) % N
    row2 = jnp.concatenate([row, padi]).reshape(EP // 128, 128)
    col2 = jnp.concatenate([col, padi]).reshape(EP // 128, 128)

    inter = params["interactions"]
    w1s = jnp.stack([blk["mlp_w1"] for blk in inter])
    b1s = jnp.stack([blk["mlp_b1"] for blk in inter]).reshape(NI, 1, F)
    w2s = jnp.stack([blk["mlp_w2"] for blk in inter])
    b2s = jnp.stack([blk["mlp_b2"] for blk in inter]).reshape(NI, 1, F)

    d2v = _dist(ptab, row2, col2)

    conv = _conv_fn()
    h, xpad = _embed(atomic_numbers.astype(jnp.int32), params["embedding"],
                     inter[0]["lin1_w"])
    lin2w_s = jnp.stack([blk["lin2_w"] for blk in inter])
    lin2b_s = jnp.stack([blk["lin2_b"] for blk in inter]).reshape(NI, 1, H)
    linw_s = jnp.stack([blk["lin_w"] for blk in inter])
    linb_s = jnp.stack([blk["lin_b"] for blk in inter]).reshape(NI, 1, H)
    lin1n_s = jnp.stack([inter[(b2 + 1) % NI]["lin1_w"] for b2 in range(NI)])
    fw1_s = jnp.stack([w1s[(b2 + 1) % NI] for b2 in range(NI)])
    fb1_s = jnp.stack([b1s[(b2 + 1) % NI] for b2 in range(NI)])
    fw2_s = jnp.stack([w2s[(b2 + 1) % NI] for b2 in range(NI)])
    fb2_s = jnp.stack([b2s[(b2 + 1) % NI] for b2 in range(NI)])

    wf0 = _filter1(d2v, w1s[0], b1s[0], w2s[0], b2s[0])

    def body(carry, xs):
        h_c, xpad_c, wf_c = carry
        fw1, fb1, fw2, fb2, l2w, l2b, lw, lb, l1n = xs
        xc = _compact(xpad_c.reshape(2 * N, 128))
        aggA, aggB = conv(xc, wf_c, row2, col2)
        wf_n = _filter1(d2v, fw1, fb1, fw2, fb2)
        h_n, xpad_n = _update(aggA, aggB, h_c, l2w, l2b, lw, lb, l1n)
        return (h_n, xpad_n, wf_n), ()

    (h, xpad, _), _ = lax.scan(
        body, (h, xpad, wf0),
        (fw1_s, fb1_s, fw2_s, fb2_s, lin2w_s, lin2b_s, linw_s, linb_s,
         lin1n_s))

    en = _final(h, params["out1_w"], params["out1_b"].reshape(1, H // 2),
                params["out2_w"], params["out2_b"].reshape(1, 1),
                batch_indices.astype(jnp.int32))
    energies = en.reshape(B)
    forces = jnp.zeros((N, 3), f32)
    stress = jnp.zeros((B, 3, 3), f32)
    features = jnp.zeros((N, H), f32)
    return (energies, forces, stress, features)
